# baseline (speedup 1.0000x reference)
import functools

import jax
import jax.numpy as jnp
from jax import lax
from jax.experimental import pallas as pl
from jax.experimental.pallas import tpu as pltpu
from jax.experimental.pallas import tpu_sc as plsc

N_NODES = 10000
N_EDGES = 160000
NUM_TILES = 16
NUM_CORES = 2
BUCKET = 640
ACC_ROWS = 648
EDGE_BLOCK = 64
GROUP = 4
STAGE_GROUPS = 10
STAGE_EDGES = STAGE_GROUPS * GROUP * EDGE_BLOCK
GB_EDGES = GROUP * EDGE_BLOCK
E_PAD = NUM_TILES * 10240
E_STORE = E_PAD + STAGE_EDGES
DUMMY_DST = N_NODES


def _sc_agg_body(n_pairs, h_ref, srcp_ref, dst_ref, gs_ref, ng_ref, init_ref,
                 out_ref, acc, src_v, dst_v, gs_v, ng_v,
                 r0, r1, r2, r3, s0, s1, s2, s3):
    cid = lax.axis_index("c")
    tid = lax.axis_index("s")
    rows = (r0, r1, r2, r3)
    sems = (s0, s1, s2, s3)
    lane = jnp.arange(16, dtype=jnp.int32)
    lane128 = lane * 128

    pltpu.sync_copy(gs_ref, gs_v)
    pltpu.sync_copy(ng_ref, ng_v)
    tmask = lane == tid
    gs = jnp.max(jnp.where(tmask, gs_v[...], 0))
    ng = jnp.max(jnp.where(tmask, ng_v[...], 0))
    ns = (ng + STAGE_GROUPS - 1) // STAGE_GROUPS

    def pair_body(p, carry0):
        pltpu.sync_copy(init_ref, acc)

        def stage_body(s, carry1):
            ebase = (gs + s * STAGE_GROUPS) * GB_EDGES
            pltpu.sync_copy(srcp_ref.at[p, cid, pl.ds(ebase, STAGE_EDGES)],
                            src_v)
            pltpu.sync_copy(dst_ref.at[pl.ds(ebase, STAGE_EDGES)], dst_v)
            glim = jnp.minimum(STAGE_GROUPS, ng - s * STAGE_GROUPS)
            for b in range(GROUP):
                pltpu.async_copy(
                    h_ref.at[src_v.at[pl.ds(b * EDGE_BLOCK, EDGE_BLOCK)]],
                    rows[b], sems[b])

            def group_body(g, carry2):
                for b in range(GROUP):
                    boff = (g * GROUP + b) * EDGE_BLOCK
                    pltpu.make_async_copy(
                        h_ref.at[src_v.at[pl.ds(boff, EDGE_BLOCK)]],
                        rows[b], sems[b]).wait()
                    for e in range(4):
                        dstv = dst_v[pl.ds(boff + e * 16, 16)]
                        dl = dstv - tid * BUCKET
                        sel = (dl >= 0) & (dl < BUCKET)
                        dl = jnp.where(sel, dl, BUCKET)
                        evv = lane + e * 16

                        def col_body(c8, cv):
                            for cu in range(16):
                                v = plsc.load_gather(rows[b], [evv, cv])
                                plsc.addupdate_scatter(acc, [dl, cv], v)
                                cv = cv + 1
                            return cv
                        lax.fori_loop(0, 8, col_body,
                                      jnp.zeros((16,), jnp.int32))

                    @pl.when(g + 1 < glim)
                    def _():
                        nboff = ((g + 1) * GROUP + b) * EDGE_BLOCK
                        pltpu.async_copy(
                            h_ref.at[src_v.at[pl.ds(nboff, EDGE_BLOCK)]],
                            rows[b], sems[b])
                return carry2
            lax.fori_loop(0, glim, group_body, 0)
            return carry1
        lax.fori_loop(0, ns, stage_body, 0)

        obase = (2 * p + cid) * N_NODES + tid * BUCKET

        @pl.when(tid < NUM_TILES - 1)
        def _():
            pltpu.sync_copy(acc.at[pl.ds(0, BUCKET)],
                            out_ref.at[pl.ds(obase, BUCKET)])

        @pl.when(tid == NUM_TILES - 1)
        def _():
            rem = N_NODES - (NUM_TILES - 1) * BUCKET
            pltpu.sync_copy(acc.at[pl.ds(0, rem)],
                            out_ref.at[pl.ds(obase, rem)])
        return carry0
    lax.fori_loop(0, n_pairs, pair_body, 0)


def _make_sc_aggregate(n_pairs):
    return functools.partial(
        pl.kernel,
        out_type=jax.ShapeDtypeStruct(
            (n_pairs * NUM_CORES * N_NODES, 128), jnp.float32),
        mesh=plsc.VectorSubcoreMesh(core_axis_name="c", subcore_axis_name="s"),
        compiler_params=pltpu.CompilerParams(needs_layout_passes=False),
        scratch_types=(
            [pltpu.VMEM((ACC_ROWS, 128), jnp.float32),
             pltpu.VMEM((STAGE_EDGES,), jnp.int32),
             pltpu.VMEM((STAGE_EDGES,), jnp.int32),
             pltpu.VMEM((16,), jnp.int32),
             pltpu.VMEM((16,), jnp.int32)]
            + [pltpu.VMEM((EDGE_BLOCK, 128), jnp.float32)] * GROUP
            + [pltpu.SemaphoreType.DMA] * GROUP
        ),
    )(functools.partial(_sc_agg_body, n_pairs))


_sc_aggregate_1 = _make_sc_aggregate(1)
_sc_aggregate_4 = _make_sc_aggregate(4)


def _mm_body(oc, bn, relu, a_ref, w_ref, b_ref, o_ref):
    c = pl.program_id(1)
    nc = pl.num_programs(1)

    @pl.when(c == 0)
    def _():
        for o in range(oc):
            o_ref[o] = jnp.broadcast_to(b_ref[o], (bn, 128))

    m = jnp.dot(a_ref[...], w_ref[...], preferred_element_type=jnp.float32)
    for o in range(oc):
        o_ref[o] += m[:, o * 128:(o + 1) * 128]

    if relu:
        @pl.when(c == nc - 1)
        def _():
            for o in range(oc):
                o_ref[o] = jnp.maximum(o_ref[o], 0.0)


def _tc_matmul(a3, w3, bias, relu):
    cc, n, _ = a3.shape
    o_full = w3.shape[2]
    oc = o_full // 128
    bn = 1000
    grid = (n // bn, cc)
    bias3 = bias.reshape(oc, 1, 128)
    return pl.pallas_call(
        functools.partial(_mm_body, oc, bn, relu),
        grid=grid,
        in_specs=[
            pl.BlockSpec((None, bn, 128), lambda nb, c: (c, nb, 0)),
            pl.BlockSpec((None, 128, o_full), lambda nb, c: (c, 0, 0)),
            pl.BlockSpec((oc, 1, 128), lambda nb, c: (0, 0, 0)),
        ],
        out_specs=pl.BlockSpec((oc, bn, 128), lambda nb, c: (0, nb, 0)),
        out_shape=jax.ShapeDtypeStruct((oc, n, 128), jnp.float32),
    )(a3, w3, bias3)


def kernel(features, edge_index, W0, b0, W1, b1, W2, b2):
    n, f_in = features.shape
    src = edge_index[0]
    dst = edge_index[1]
    pad = E_PAD - N_EDGES
    src_p = jnp.concatenate([src, jnp.zeros((pad,), jnp.int32)])
    dst_p = jnp.concatenate([dst, jnp.full((pad,), DUMMY_DST, jnp.int32)])
    key = dst_p // BUCKET
    perm = jnp.argsort(key, stable=True)
    src_s = src_p[perm]
    dst_s = dst_p[perm]
    key_s = key[perm]
    tail = E_STORE - E_PAD
    src_s = jnp.concatenate([src_s, jnp.zeros((tail,), jnp.int32)])
    dst_s = jnp.concatenate([dst_s, jnp.full((tail,), DUMMY_DST, jnp.int32)])
    starts = jnp.searchsorted(key_s, jnp.arange(17, dtype=jnp.int32))
    starts = starts.astype(jnp.int32)
    gs = starts[:16] // GB_EDGES
    ge = (starts[1:] + GB_EDGES - 1) // GB_EDGES
    ng = (ge - gs).astype(jnp.int32)
    gs = gs.astype(jnp.int32)
    chunk_off = (jnp.arange(4)[:, None] * 2 + jnp.arange(2)[None, :]) * n
    srcp_all = src_s[None, None, :] + chunk_off[:, :, None]
    srcp_1 = srcp_all[:1]
    zero_init = jnp.zeros((ACC_ROWS, 128), jnp.float32)
    b2_init = jnp.broadcast_to(b2, (ACC_ROWS, 128))

    x3 = features.reshape(n, 2, 128).transpose(1, 0, 2)
    a0 = _sc_aggregate_1(x3.reshape(2 * n, 128), srcp_1, dst_s, gs, ng,
                         zero_init)
    a0 = a0.reshape(2, n, 128)
    h1 = _tc_matmul(a0, W0.reshape(2, 128, -1), b0, relu=True)

    a1 = _sc_aggregate_4(h1.reshape(8 * n, 128), srcp_all, dst_s, gs, ng,
                         zero_init)
    a1 = a1.reshape(8, n, 128)
    h2 = _tc_matmul(a1, W1.reshape(8, 128, -1), b1, relu=True)

    t = _tc_matmul(h2, W2.reshape(8, 128, -1), jnp.zeros((128,), jnp.float32),
                   relu=False)
    tcat = jnp.concatenate([t[0], t[0]], axis=0)
    out = _sc_aggregate_1(tcat, srcp_1, dst_s, gs, ng, b2_init)
    return out[:n]


# speedup vs baseline: 5.8916x; 5.8916x over previous
import functools

import jax
import jax.numpy as jnp
from jax import lax
from jax.experimental import pallas as pl
from jax.experimental.pallas import tpu as pltpu
from jax.experimental.pallas import tpu_sc as plsc

N_NODES = 10000
N_EDGES = 160000
NUM_TILES = 16
NUM_CORES = 2
EDGE_BLOCK = 128
EDGES_PER_TILE = 10240
N_HALVES = 2
NB_H = EDGES_PER_TILE // (N_HALVES * EDGE_BLOCK)
E_PAD = NUM_TILES * EDGES_PER_TILE
ACC_ROWS = 10112
INIT_ROWS = ACC_ROWS // NUM_TILES
OUT_ROWS = 624
DUMMY_ROW = N_NODES


def _agg_window(h_ref, acc, src_v, dst_v, rows_a, rows_b, sem_a, sem_b):
    pltpu.async_copy(h_ref.at[src_v.at[0]], rows_a, sem_a)

    def body(i, carry):
        j = 2 * i
        pltpu.async_copy(h_ref.at[src_v.at[j + 1]], rows_b, sem_b)
        pltpu.make_async_copy(h_ref.at[src_v.at[j]], rows_a, sem_a).wait()
        pltpu.sync_copy(rows_a, acc.at[dst_v.at[j]], add=True)
        pltpu.async_copy(h_ref.at[src_v.at[j + 2]], rows_a, sem_a)
        pltpu.make_async_copy(h_ref.at[src_v.at[j + 1]], rows_b, sem_b).wait()
        pltpu.sync_copy(rows_b, acc.at[dst_v.at[j + 1]], add=True)
        return carry
    lax.fori_loop(0, (NB_H - 2) // 2, body, 0)
    j = NB_H - 2
    pltpu.async_copy(h_ref.at[src_v.at[j + 1]], rows_b, sem_b)
    pltpu.make_async_copy(h_ref.at[src_v.at[j]], rows_a, sem_a).wait()
    pltpu.sync_copy(rows_a, acc.at[dst_v.at[j]], add=True)
    pltpu.make_async_copy(h_ref.at[src_v.at[j + 1]], rows_b, sem_b).wait()
    pltpu.sync_copy(rows_b, acc.at[dst_v.at[j + 1]], add=True)


def _flush(acc, out_ref, cid, tid):
    obase = cid * N_NODES

    @pl.when(tid < NUM_TILES - 1)
    def _():
        pltpu.sync_copy(
            acc.at[pl.ds(tid * OUT_ROWS, OUT_ROWS)],
            out_ref.at[pl.ds(obase + tid * OUT_ROWS, OUT_ROWS)])

    @pl.when(tid == NUM_TILES - 1)
    def _():
        last = (NUM_TILES - 1) * OUT_ROWS
        pltpu.sync_copy(
            acc.at[pl.ds(last, N_NODES - last)],
            out_ref.at[pl.ds(obase + last, N_NODES - last)])


_SC_SCRATCH = [
    pltpu.VMEM_SHARED((ACC_ROWS, 128), jnp.float32),
    pltpu.VMEM((NB_H, EDGE_BLOCK), jnp.int32),
    pltpu.VMEM((NB_H, EDGE_BLOCK), jnp.int32),
    pltpu.VMEM((EDGE_BLOCK, 128), jnp.float32),
    pltpu.VMEM((EDGE_BLOCK, 128), jnp.float32),
    pltpu.SemaphoreType.DMA,
    pltpu.SemaphoreType.DMA,
]
_SC_MESH = plsc.VectorSubcoreMesh(core_axis_name="c", subcore_axis_name="s")


@functools.partial(
    pl.kernel,
    out_type=jax.ShapeDtypeStruct((NUM_CORES * N_NODES, 128), jnp.float32),
    mesh=_SC_MESH, scratch_types=_SC_SCRATCH)
def _sc_agg_pair(h_ref, srcp_ref, dst_ref, init_ref, out_ref,
                 acc, src_v, dst_v, rows_a, rows_b, sem_a, sem_b):
    cid = lax.axis_index("c")
    tid = lax.axis_index("s")
    pltpu.sync_copy(init_ref, acc.at[pl.ds(tid * INIT_ROWS, INIT_ROWS)])
    plsc.subcore_barrier()
    for half in range(N_HALVES):
        pltpu.sync_copy(srcp_ref.at[cid, tid, half], src_v)
        pltpu.sync_copy(dst_ref.at[tid, half], dst_v)
        _agg_window(h_ref, acc, src_v, dst_v, rows_a, rows_b, sem_a, sem_b)
    plsc.subcore_barrier()
    _flush(acc, out_ref, cid, tid)


@functools.partial(
    pl.kernel,
    out_type=jax.ShapeDtypeStruct((NUM_CORES * N_NODES, 128), jnp.float32),
    mesh=_SC_MESH, scratch_types=_SC_SCRATCH)
def _sc_agg_split(h_ref, srcs_ref, dsts_ref, init_ref, out_ref,
                  acc, src_v, dst_v, rows_a, rows_b, sem_a, sem_b):
    cid = lax.axis_index("c")
    tid = lax.axis_index("s")
    pltpu.sync_copy(init_ref.at[cid],
                    acc.at[pl.ds(tid * INIT_ROWS, INIT_ROWS)])
    plsc.subcore_barrier()
    pltpu.sync_copy(srcs_ref.at[cid, tid], src_v)
    pltpu.sync_copy(dsts_ref.at[cid, tid], dst_v)
    _agg_window(h_ref, acc, src_v, dst_v, rows_a, rows_b, sem_a, sem_b)
    plsc.subcore_barrier()
    _flush(acc, out_ref, cid, tid)


def _mm_body(oc, bn, relu, a_ref, w_ref, b_ref, o_ref):
    c = pl.program_id(1)
    nc = pl.num_programs(1)

    @pl.when(c == 0)
    def _():
        for o in range(oc):
            o_ref[o] = jnp.broadcast_to(b_ref[o], (bn, 128))

    m = jnp.dot(a_ref[...].astype(jnp.bfloat16),
                w_ref[...].astype(jnp.bfloat16),
                preferred_element_type=jnp.float32)
    for o in range(oc):
        o_ref[o] += m[:, o * 128:(o + 1) * 128]

    if relu:
        @pl.when(c == nc - 1)
        def _():
            for o in range(oc):
                o_ref[o] = jnp.maximum(o_ref[o], 0.0)


def _tc_matmul(a3, w3, bias, relu):
    cc, n, _ = a3.shape
    o_full = w3.shape[2]
    oc = o_full // 128
    bn = 1000
    return pl.pallas_call(
        functools.partial(_mm_body, oc, bn, relu),
        grid=(n // bn, cc),
        in_specs=[
            pl.BlockSpec((None, bn, 128), lambda nb, c: (c, nb, 0)),
            pl.BlockSpec((None, 128, o_full), lambda nb, c: (c, 0, 0)),
            pl.BlockSpec((oc, 1, 128), lambda nb, c: (0, 0, 0)),
        ],
        out_specs=pl.BlockSpec((oc, bn, 128), lambda nb, c: (0, nb, 0)),
        out_shape=jax.ShapeDtypeStruct((oc, n, 128), jnp.float32),
    )(a3, w3, bias.reshape(oc, 1, 128))


def _mm_acc_body(oc, bn, first, last, a_ref, w_ref, b_ref, z_ref, o_ref):
    c = pl.program_id(1)
    nc = pl.num_programs(1)

    @pl.when(c == 0)
    def _():
        for o in range(oc):
            if first:
                o_ref[o] = jnp.broadcast_to(b_ref[o], (bn, 128))
            else:
                o_ref[o] = z_ref[o]

    m = jnp.dot(a_ref[...].astype(jnp.bfloat16),
                w_ref[...].astype(jnp.bfloat16),
                preferred_element_type=jnp.float32)
    for o in range(oc):
        o_ref[o] += m[:, o * 128:(o + 1) * 128]

    if last:
        @pl.when(c == nc - 1)
        def _():
            for o in range(oc):
                o_ref[o] = jnp.maximum(o_ref[o], 0.0)


def _tc_matmul_acc(a3, w3, bias, z, first, last):
    cc, n, _ = a3.shape
    o_full = w3.shape[2]
    oc = o_full // 128
    bn = 1000
    if z is None:
        z = jnp.zeros((oc, n, 128), jnp.float32)
    return pl.pallas_call(
        functools.partial(_mm_acc_body, oc, bn, first, last),
        grid=(n // bn, cc),
        in_specs=[
            pl.BlockSpec((None, bn, 128), lambda nb, c: (c, nb, 0)),
            pl.BlockSpec((None, 128, o_full), lambda nb, c: (c, 0, 0)),
            pl.BlockSpec((oc, 1, 128), lambda nb, c: (0, 0, 0)),
            pl.BlockSpec((oc, bn, 128), lambda nb, c: (0, nb, 0)),
        ],
        out_specs=pl.BlockSpec((oc, bn, 128), lambda nb, c: (0, nb, 0)),
        out_shape=jax.ShapeDtypeStruct((oc, n, 128), jnp.float32),
    )(a3, w3, bias.reshape(oc, 1, 128), z)


def _add_body(pa_ref, pb_ref, o_ref):
    o_ref[...] = pa_ref[...] + pb_ref[...]


def _tc_add(pa, pb):
    n = pa.shape[0]
    bn = 1000
    return pl.pallas_call(
        _add_body,
        grid=(n // bn,),
        in_specs=[pl.BlockSpec((bn, 128), lambda nb: (nb, 0))] * 2,
        out_specs=pl.BlockSpec((bn, 128), lambda nb: (nb, 0)),
        out_shape=jax.ShapeDtypeStruct((n, 128), jnp.float32),
    )(pa, pb)


def kernel(features, edge_index, W0, b0, W1, b1, W2, b2):
    n, f_in = features.shape
    src = edge_index[0]
    dst = edge_index[1]
    pad = E_PAD - N_EDGES
    src_p = jnp.concatenate([src, jnp.zeros((pad,), jnp.int32)])
    dst_p = jnp.concatenate([dst, jnp.full((pad,), DUMMY_ROW, jnp.int32)])
    chunk_off = (jnp.arange(4)[:, None] * 2 + jnp.arange(2)[None, :]) * n
    srcp_all = (src_p[None, None, :] + chunk_off[:, :, None]).reshape(
        4, NUM_CORES, NUM_TILES, N_HALVES, NB_H, EDGE_BLOCK)
    dst3 = dst_p.reshape(NUM_TILES, N_HALVES, NB_H, EDGE_BLOCK)
    srcs = src_p.reshape(NUM_CORES, NUM_TILES, NB_H, EDGE_BLOCK)
    dsts = dst_p.reshape(NUM_CORES, NUM_TILES, NB_H, EDGE_BLOCK)
    zero_init = jnp.zeros((INIT_ROWS, 128), jnp.float32)
    b2_init = jnp.stack([jnp.broadcast_to(b2, (INIT_ROWS, 128)),
                         jnp.zeros((INIT_ROWS, 128), jnp.float32)])

    x3 = features.reshape(n, 2, 128).transpose(1, 0, 2)
    a0 = _sc_agg_pair(x3.reshape(2 * n, 128), srcp_all[0], dst3, zero_init)
    a0 = a0.reshape(2, n, 128)
    h1 = _tc_matmul(a0, W0.reshape(2, 128, -1), b0, relu=True)

    w1r = W1.reshape(8, 128, -1)
    z = None
    for p in range(4):
        ap = _sc_agg_pair(h1[2 * p:2 * p + 2].reshape(2 * n, 128),
                          srcp_all[p], dst3, zero_init)
        z = _tc_matmul_acc(ap.reshape(2, n, 128), w1r[2 * p:2 * p + 2],
                           b1, z, first=(p == 0), last=(p == 3))
    h2 = z

    t = _tc_matmul(h2, W2.reshape(8, 128, -1), jnp.zeros((128,), jnp.float32),
                   relu=False)
    parts = _sc_agg_split(t[0], srcs, dsts, b2_init)
    return _tc_add(parts[:n], parts[n:])


# speedup vs baseline: 6.0056x; 1.0193x over previous
import functools

import jax
import jax.numpy as jnp
from jax import lax
from jax.experimental import pallas as pl
from jax.experimental.pallas import tpu as pltpu
from jax.experimental.pallas import tpu_sc as plsc

N_NODES = 10000
N_EDGES = 160000
NUM_TILES = 16
NUM_CORES = 2
EDGE_BLOCK = 128
EDGES_PER_TILE = 10240
N_HALVES = 2
NB_H = EDGES_PER_TILE // (N_HALVES * EDGE_BLOCK)
E_PAD = NUM_TILES * EDGES_PER_TILE
ACC_ROWS = 10112
INIT_ROWS = ACC_ROWS // NUM_TILES
OUT_ROWS = 624
DUMMY_ROW = N_NODES


def _agg_window(h_ref, acc, src_v, dst_v, rows_a, rows_b, sem_a, sem_b):
    pltpu.async_copy(h_ref.at[src_v.at[0]], rows_a, sem_a)

    def body(i, carry):
        j = 2 * i
        pltpu.async_copy(h_ref.at[src_v.at[j + 1]], rows_b, sem_b)
        pltpu.make_async_copy(h_ref.at[src_v.at[j]], rows_a, sem_a).wait()
        pltpu.sync_copy(rows_a, acc.at[dst_v.at[j]], add=True)
        pltpu.async_copy(h_ref.at[src_v.at[j + 2]], rows_a, sem_a)
        pltpu.make_async_copy(h_ref.at[src_v.at[j + 1]], rows_b, sem_b).wait()
        pltpu.sync_copy(rows_b, acc.at[dst_v.at[j + 1]], add=True)
        return carry
    lax.fori_loop(0, (NB_H - 2) // 2, body, 0)
    j = NB_H - 2
    pltpu.async_copy(h_ref.at[src_v.at[j + 1]], rows_b, sem_b)
    pltpu.make_async_copy(h_ref.at[src_v.at[j]], rows_a, sem_a).wait()
    pltpu.sync_copy(rows_a, acc.at[dst_v.at[j]], add=True)
    pltpu.make_async_copy(h_ref.at[src_v.at[j + 1]], rows_b, sem_b).wait()
    pltpu.sync_copy(rows_b, acc.at[dst_v.at[j + 1]], add=True)


def _flush(acc, out_ref, cid, tid):
    obase = cid * N_NODES

    @pl.when(tid < NUM_TILES - 1)
    def _():
        pltpu.sync_copy(
            acc.at[pl.ds(tid * OUT_ROWS, OUT_ROWS)],
            out_ref.at[pl.ds(obase + tid * OUT_ROWS, OUT_ROWS)])

    @pl.when(tid == NUM_TILES - 1)
    def _():
        last = (NUM_TILES - 1) * OUT_ROWS
        pltpu.sync_copy(
            acc.at[pl.ds(last, N_NODES - last)],
            out_ref.at[pl.ds(obase + last, N_NODES - last)])


_SC_SCRATCH = [
    pltpu.VMEM_SHARED((ACC_ROWS, 128), jnp.float32),
    pltpu.VMEM((NB_H, EDGE_BLOCK), jnp.int32),
    pltpu.VMEM((NB_H, EDGE_BLOCK), jnp.int32),
    pltpu.VMEM((EDGE_BLOCK, 128), jnp.float32),
    pltpu.VMEM((EDGE_BLOCK, 128), jnp.float32),
    pltpu.SemaphoreType.DMA,
    pltpu.SemaphoreType.DMA,
]
_SC_MESH = plsc.VectorSubcoreMesh(core_axis_name="c", subcore_axis_name="s")


@functools.partial(
    pl.kernel,
    out_type=jax.ShapeDtypeStruct((NUM_CORES * N_NODES, 128), jnp.float32),
    mesh=_SC_MESH, scratch_types=_SC_SCRATCH)
def _sc_agg_pair(h_ref, srcp_ref, dst_ref, init_ref, out_ref,
                 acc, src_v, dst_v, rows_a, rows_b, sem_a, sem_b):
    cid = lax.axis_index("c")
    tid = lax.axis_index("s")
    pltpu.sync_copy(init_ref, acc.at[pl.ds(tid * INIT_ROWS, INIT_ROWS)])
    plsc.subcore_barrier()
    for half in range(N_HALVES):
        pltpu.sync_copy(srcp_ref.at[cid, tid, half], src_v)
        pltpu.sync_copy(dst_ref.at[tid, half], dst_v)
        _agg_window(h_ref, acc, src_v, dst_v, rows_a, rows_b, sem_a, sem_b)
    plsc.subcore_barrier()
    _flush(acc, out_ref, cid, tid)


@functools.partial(
    pl.kernel,
    out_type=jax.ShapeDtypeStruct((NUM_CORES * N_NODES, 128), jnp.float32),
    mesh=_SC_MESH, scratch_types=_SC_SCRATCH)
def _sc_agg_split(h_ref, srcs_ref, dsts_ref, init_ref, out_ref,
                  acc, src_v, dst_v, rows_a, rows_b, sem_a, sem_b):
    cid = lax.axis_index("c")
    tid = lax.axis_index("s")
    pltpu.sync_copy(init_ref.at[cid],
                    acc.at[pl.ds(tid * INIT_ROWS, INIT_ROWS)])
    plsc.subcore_barrier()
    pltpu.sync_copy(srcs_ref.at[cid, tid], src_v)
    pltpu.sync_copy(dsts_ref.at[cid, tid], dst_v)
    _agg_window(h_ref, acc, src_v, dst_v, rows_a, rows_b, sem_a, sem_b)
    plsc.subcore_barrier()
    _flush(acc, out_ref, cid, tid)


def _mm_body(oc, bn, relu, a_ref, w_ref, b_ref, o_ref):
    c = pl.program_id(1)
    nc = pl.num_programs(1)

    @pl.when(c == 0)
    def _():
        for o in range(oc):
            o_ref[o] = jnp.broadcast_to(b_ref[o], (bn, 128))

    m = jnp.dot(a_ref[...].astype(jnp.bfloat16),
                w_ref[...].astype(jnp.bfloat16),
                preferred_element_type=jnp.float32)
    for o in range(oc):
        o_ref[o] += m[:, o * 128:(o + 1) * 128]

    if relu:
        @pl.when(c == nc - 1)
        def _():
            for o in range(oc):
                o_ref[o] = jnp.maximum(o_ref[o], 0.0)


def _tc_matmul(a3, w3, bias, relu):
    cc, n, _ = a3.shape
    o_full = w3.shape[2]
    oc = o_full // 128
    bn = 1000
    return pl.pallas_call(
        functools.partial(_mm_body, oc, bn, relu),
        grid=(n // bn, cc),
        in_specs=[
            pl.BlockSpec((None, bn, 128), lambda nb, c: (c, nb, 0)),
            pl.BlockSpec((None, 128, o_full), lambda nb, c: (c, 0, 0)),
            pl.BlockSpec((oc, 1, 128), lambda nb, c: (0, 0, 0)),
        ],
        out_specs=pl.BlockSpec((oc, bn, 128), lambda nb, c: (0, nb, 0)),
        out_shape=jax.ShapeDtypeStruct((oc, n, 128), jnp.float32),
    )(a3, w3, bias.reshape(oc, 1, 128))


def _mm_acc_body(oc, bn, first, last, a_ref, w_ref, b_ref, z_ref, o_ref):
    c = pl.program_id(1)
    nc = pl.num_programs(1)

    @pl.when(c == 0)
    def _():
        for o in range(oc):
            if first:
                o_ref[o] = jnp.broadcast_to(b_ref[o], (bn, 128))
            else:
                o_ref[o] = z_ref[o]

    m = jnp.dot(a_ref[...].astype(jnp.bfloat16),
                w_ref[...].astype(jnp.bfloat16),
                preferred_element_type=jnp.float32)
    for o in range(oc):
        o_ref[o] += m[:, o * 128:(o + 1) * 128]

    if last:
        @pl.when(c == nc - 1)
        def _():
            for o in range(oc):
                o_ref[o] = jnp.maximum(o_ref[o], 0.0)


def _tc_matmul_acc(a3, w3, bias, z, first, last):
    cc, n, _ = a3.shape
    o_full = w3.shape[2]
    oc = o_full // 128
    bn = 1000
    if z is None:
        z = jnp.zeros((oc, n, 128), jnp.float32)
    return pl.pallas_call(
        functools.partial(_mm_acc_body, oc, bn, first, last),
        grid=(n // bn, cc),
        in_specs=[
            pl.BlockSpec((None, bn, 128), lambda nb, c: (c, nb, 0)),
            pl.BlockSpec((None, 128, o_full), lambda nb, c: (c, 0, 0)),
            pl.BlockSpec((oc, 1, 128), lambda nb, c: (0, 0, 0)),
            pl.BlockSpec((oc, bn, 128), lambda nb, c: (0, nb, 0)),
        ],
        out_specs=pl.BlockSpec((oc, bn, 128), lambda nb, c: (0, nb, 0)),
        out_shape=jax.ShapeDtypeStruct((oc, n, 128), jnp.float32),
    )(a3, w3, bias.reshape(oc, 1, 128), z)


def _add_body(pa_ref, pb_ref, o_ref):
    o_ref[...] = pa_ref[...] + pb_ref[...]


def _tc_add(pa, pb):
    n = pa.shape[0]
    bn = 1000
    return pl.pallas_call(
        _add_body,
        grid=(n // bn,),
        in_specs=[pl.BlockSpec((bn, 128), lambda nb: (nb, 0))] * 2,
        out_specs=pl.BlockSpec((bn, 128), lambda nb: (nb, 0)),
        out_shape=jax.ShapeDtypeStruct((n, 128), jnp.float32),
    )(pa, pb)


def kernel(features, edge_index, W0, b0, W1, b1, W2, b2):
    n, f_in = features.shape
    src = edge_index[0]
    dst = edge_index[1]
    pad = E_PAD - N_EDGES
    src_p = jnp.concatenate([src, jnp.zeros((pad,), jnp.int32)])
    dst_p = jnp.concatenate([dst, jnp.full((pad,), DUMMY_ROW, jnp.int32)])
    chunk_off = (jnp.arange(4)[:, None] * 2 + jnp.arange(2)[None, :]) * n
    srcp_all = (src_p[None, None, :] + chunk_off[:, :, None]).reshape(
        4, NUM_CORES, NUM_TILES, N_HALVES, NB_H, EDGE_BLOCK)
    dst3 = dst_p.reshape(NUM_TILES, N_HALVES, NB_H, EDGE_BLOCK)
    srcs = src_p.reshape(NUM_CORES, NUM_TILES, NB_H, EDGE_BLOCK)
    dsts = dst_p.reshape(NUM_CORES, NUM_TILES, NB_H, EDGE_BLOCK)
    zero_init = jnp.zeros((INIT_ROWS, 128), jnp.float32)
    b2_init = jnp.stack([jnp.broadcast_to(b2, (INIT_ROWS, 128)),
                         jnp.zeros((INIT_ROWS, 128), jnp.float32)])

    x3 = features.reshape(n, 2, 128).transpose(1, 0, 2)
    a0 = _sc_agg_pair(x3.reshape(2 * n, 128), srcp_all[0], dst3, zero_init)
    a0 = a0.reshape(2, n, 128)
    h1 = _tc_matmul(a0, W0.reshape(2, 128, -1), b0, relu=True)

    w1r = W1.reshape(8, 128, -1)
    z = None
    for p in range(4):
        ap = _sc_agg_pair(h1.reshape(8 * n, 128),
                          srcp_all[p], dst3, zero_init)
        z = _tc_matmul_acc(ap.reshape(2, n, 128), w1r[2 * p:2 * p + 2],
                           b1, z, first=(p == 0), last=(p == 3))
    h2 = z

    t = _tc_matmul(h2, W2.reshape(8, 128, -1), jnp.zeros((128,), jnp.float32),
                   relu=False)
    parts = _sc_agg_split(t[0], srcs, dsts, b2_init)
    return _tc_add(parts[:n], parts[n:])
